# Initial kernel scaffold; baseline (speedup 1.0000x reference)
import jax
import jax.numpy as jnp
from jax.experimental import pallas as pl


def kernel(x, kmvalue):
    raise NotImplementedError("write your pallas kernel here")


# speedup vs baseline: 1.9917x; 1.9917x over previous
import functools

import jax
import jax.numpy as jnp
from jax import lax
from jax.experimental import pallas as pl
from jax.experimental.pallas import tpu as pltpu
from jax.experimental.pallas import tpu_sc as plsc

N = 8 * 1024 * 384
NC, NS, L = 2, 16, 16
NW = NC * NS
PER_W = N // NW
CHUNK = 32768
N_CHUNKS = PER_W // CHUNK
NV = CHUNK // L


def _body(x_hbm, kmv_hbm, out_hbm, kmv_v, buf, sem):
    wid = lax.axis_index("s") * NC + lax.axis_index("c")
    base = wid * PER_W

    pltpu.sync_copy(kmv_hbm, kmv_v)
    snd, _ = plsc.sort_key_val(kmv_v[...], lax.iota(jnp.int32, 16))

    sv = [snd[i] for i in range(16)]
    vb = [jnp.full((L,), sv[i], dtype=jnp.float32) for i in range(16)]
    mb = [jnp.full((L,), 0.5 * (sv[i] + sv[i + 1]), dtype=jnp.float32)
          for i in range(15)]

    def do_chunk(c, _):
        off = base + c * CHUNK
        pltpu.sync_copy(x_hbm.at[pl.ds(off, CHUNK)], buf)

        def quant_vreg(i, _):
            xv = buf[pl.ds(i * L, L)]
            r = vb[0]
            for k in range(15):
                r = jnp.where(xv > mb[k], vb[k + 1], r)
            buf[pl.ds(i * L, L)] = r
            return 0

        lax.fori_loop(0, NV, quant_vreg, 0)
        pltpu.sync_copy(buf, out_hbm.at[pl.ds(off, CHUNK)])
        return 0

    lax.fori_loop(0, N_CHUNKS, do_chunk, 0)


@jax.jit
def _quantize(x_flat, kmvalue):
    mesh = plsc.VectorSubcoreMesh(core_axis_name="c", subcore_axis_name="s")
    return pl.kernel(
        _body,
        out_type=jax.ShapeDtypeStruct((N,), jnp.float32),
        mesh=mesh,
        scratch_types=[
            pltpu.VMEM((16,), jnp.float32),
            pltpu.VMEM((CHUNK,), jnp.float32),
            pltpu.SemaphoreType.DMA,
        ],
        compiler_params=pltpu.CompilerParams(needs_layout_passes=False),
    )(x_flat, kmvalue)


def kernel(x, kmvalue):
    out = _quantize(x.reshape(-1), kmvalue)
    return out.reshape(x.shape).astype(jnp.float32)


# speedup vs baseline: 3.0716x; 1.5422x over previous
import functools

import jax
import jax.numpy as jnp
from jax import lax
from jax.experimental import pallas as pl
from jax.experimental.pallas import tpu as pltpu
from jax.experimental.pallas import tpu_sc as plsc

N = 8 * 1024 * 384
NC, NS, L = 2, 16, 16
NW = NC * NS
PER_W = N // NW
N_CHUNKS = 3
CHUNK = PER_W // N_CHUNKS
NV = CHUNK // L
UNROLL = 8


def _body(x_hbm, kmv_hbm, out_hbm, kmv_v, bufs, lsems, ssems):
    wid = lax.axis_index("s") * NC + lax.axis_index("c")
    base = wid * PER_W

    loads = []
    for c in range(N_CHUNKS):
        cp = pltpu.make_async_copy(
            x_hbm.at[pl.ds(base + c * CHUNK, CHUNK)], bufs[c], lsems[c])
        cp.start()
        loads.append(cp)

    pltpu.sync_copy(kmv_hbm, kmv_v)
    snd, _ = plsc.sort_key_val(kmv_v[...], lax.iota(jnp.int32, 16))

    sv = [snd[i] for i in range(16)]
    vb = [jnp.full((L,), sv[i], dtype=jnp.float32) for i in range(16)]
    mb = [jnp.full((L,), 0.5 * (sv[i] + sv[i + 1]), dtype=jnp.float32)
          for i in range(15)]

    stores = []
    for c in range(N_CHUNKS):
        loads[c].wait()
        buf = bufs[c]

        @plsc.parallel_loop(0, NV, unroll=UNROLL)
        def quant_vreg(i):
            xv = buf[pl.ds(i * L, L)]
            r = vb[0]
            for k in range(15):
                r = jnp.where(xv > mb[k], vb[k + 1], r)
            buf[pl.ds(i * L, L)] = r

        cp = pltpu.make_async_copy(
            buf, out_hbm.at[pl.ds(base + c * CHUNK, CHUNK)], ssems[c])
        cp.start()
        stores.append(cp)

    for cp in stores:
        cp.wait()


@jax.jit
def _quantize(x_flat, kmvalue):
    mesh = plsc.VectorSubcoreMesh(core_axis_name="c", subcore_axis_name="s")
    return pl.kernel(
        _body,
        out_type=jax.ShapeDtypeStruct((N,), jnp.float32),
        mesh=mesh,
        scratch_types=[
            pltpu.VMEM((16,), jnp.float32),
            [pltpu.VMEM((CHUNK,), jnp.float32) for _ in range(N_CHUNKS)],
            [pltpu.SemaphoreType.DMA for _ in range(N_CHUNKS)],
            [pltpu.SemaphoreType.DMA for _ in range(N_CHUNKS)],
        ],
        compiler_params=pltpu.CompilerParams(needs_layout_passes=False),
    )(x_flat, kmvalue)


def kernel(x, kmvalue):
    out = _quantize(x.reshape(-1), kmvalue)
    return out.reshape(x.shape).astype(jnp.float32)


# speedup vs baseline: 3.7189x; 1.2108x over previous
import functools

import jax
import jax.numpy as jnp
from jax import lax
from jax.experimental import pallas as pl
from jax.experimental.pallas import tpu as pltpu
from jax.experimental.pallas import tpu_sc as plsc

N = 8 * 1024 * 384
NC, NS, L = 2, 16, 16
NW = NC * NS
PER_W = N // NW
N_CHUNKS = 3
CHUNK = PER_W // N_CHUNKS
NV = CHUNK // L
UNROLL = 8


def _body(x_hbm, kmv_hbm, out_hbm, kmv_v, bufs, lsems, ssems):
    wid = lax.axis_index("s") * NC + lax.axis_index("c")
    base = wid * PER_W

    loads = []
    for c in range(N_CHUNKS):
        cp = pltpu.make_async_copy(
            x_hbm.at[pl.ds(base + c * CHUNK, CHUNK)], bufs[c], lsems[c])
        cp.start()
        loads.append(cp)

    pltpu.sync_copy(kmv_hbm, kmv_v)
    snd, _ = plsc.sort_key_val(kmv_v[...], lax.iota(jnp.int32, 16))

    iota = lax.iota(jnp.int32, L)
    shifted = jnp.take_along_axis(snd, jnp.minimum(iota + 1, 15), axis=0)
    mv = 0.5 * (snd + shifted)
    mb7 = jnp.full((L,), mv[7], dtype=jnp.float32)
    i8 = jnp.full((L,), 8, dtype=jnp.int32)
    i0 = jnp.zeros((L,), dtype=jnp.int32)
    stepv = {s: jnp.full((L,), s, dtype=jnp.int32) for s in (4, 2, 1)}

    stores = []
    for c in range(N_CHUNKS):
        loads[c].wait()
        buf = bufs[c]

        @plsc.parallel_loop(0, NV, unroll=UNROLL)
        def quant_vreg(i):
            xv = buf[pl.ds(i * L, L)]
            j = jnp.where(xv > mb7, i8, i0)
            for s in (4, 2, 1):
                b = jnp.take_along_axis(mv, j + (s - 1), axis=0)
                j = j + jnp.where(xv > b, stepv[s], i0)
            buf[pl.ds(i * L, L)] = jnp.take_along_axis(snd, j, axis=0)

        cp = pltpu.make_async_copy(
            buf, out_hbm.at[pl.ds(base + c * CHUNK, CHUNK)], ssems[c])
        cp.start()
        stores.append(cp)

    for cp in stores:
        cp.wait()


@jax.jit
def _quantize(x_flat, kmvalue):
    mesh = plsc.VectorSubcoreMesh(core_axis_name="c", subcore_axis_name="s")
    return pl.kernel(
        _body,
        out_type=jax.ShapeDtypeStruct((N,), jnp.float32),
        mesh=mesh,
        scratch_types=[
            pltpu.VMEM((16,), jnp.float32),
            [pltpu.VMEM((CHUNK,), jnp.float32) for _ in range(N_CHUNKS)],
            [pltpu.SemaphoreType.DMA for _ in range(N_CHUNKS)],
            [pltpu.SemaphoreType.DMA for _ in range(N_CHUNKS)],
        ],
        compiler_params=pltpu.CompilerParams(needs_layout_passes=False),
    )(x_flat, kmvalue)


def kernel(x, kmvalue):
    out = _quantize(x.reshape(-1), kmvalue)
    return out.reshape(x.shape).astype(jnp.float32)
